# baseline (speedup 1.0000x reference)
import functools

import jax
import jax.numpy as jnp
from jax import lax
from jax.experimental import pallas as pl
from jax.experimental.pallas import tpu as pltpu
from jax.experimental.pallas import tpu_sc as plsc

N = 10000
E = 320000
F = 128
AVG_D_LOG = 3.4657359027997265
EPS = 1e-5

NPAD = 10240
NW = 32
NB = NPAD // NW
G = 128

CEA = 6400
NCHA = E // CEA
CPW = 4
NBA = CEA // G
NEXP = NPAD // 16

CEB = 4000
NCHB = E // CEB
NGB = CEB // 16
SEL = 1152

_F32MAX = 3.4028235e38

_MESH = plsc.VectorSubcoreMesh(core_axis_name="c", subcore_axis_name="s")
_CPARAMS = pltpu.CompilerParams(needs_layout_passes=False)


def _sumsq_body(hn_hbm, dst_hbm, src_hbm,
                sum_out, sq_out, deg_out,
                dst_v, src_v, rows_v, idx_stage, ones_v, zdeg_v,
                shared_acc, shared_deg, sem):
    c = lax.axis_index("c")
    s = lax.axis_index("s")

    zero16f = jnp.zeros((16,), jnp.float32)

    def _zero_rows(r, carry):
        for k in range(8):
            rows_v[r, pl.ds(k * 16, 16)] = zero16f
        return carry
    lax.fori_loop(0, G, _zero_rows, 0)
    for t in range(NEXP // G):
        pltpu.sync_copy(rows_v, shared_acc.at[pl.ds(s * NEXP + t * G, G)])
    for k in range(G // 16):
        ones_v[pl.ds(k * 16, 16)] = zero16f + 1.0
    for k in range(NEXP // 16):
        zdeg_v[pl.ds(k * 16, 16)] = zero16f
    pltpu.sync_copy(zdeg_v, shared_deg.at[pl.ds(s * NEXP, NEXP)])
    plsc.subcore_barrier()

    def _chunk(t, carry):
        ci = s + t * 16

        @pl.when(ci < NCHA)
        def _():
            pltpu.sync_copy(dst_hbm.at[pl.ds(ci * CEA, CEA)], dst_v)
            pltpu.sync_copy(src_hbm.at[pl.ds(ci * CEA, CEA)], src_v)

            def _batch(b, carry2):
                jb = b * G
                pltpu.async_copy(
                    hn_hbm.at[src_v.at[pl.ds(jb, G)]], rows_v, sem).wait()
                for k in range(G // 16):
                    idx_stage[0, pl.ds(k * 16, 16)] = dst_v[pl.ds(jb + k * 16, 16)]

                @pl.when(c == 1)
                def _():
                    def _sqr(r, carry3):
                        for k in range(8):
                            row = rows_v[r, pl.ds(k * 16, 16)]
                            rows_v[r, pl.ds(k * 16, 16)] = row * row
                        return carry3
                    lax.fori_loop(0, G, _sqr, 0)

                pltpu.sync_copy(rows_v, shared_acc.at[idx_stage.at[0]], add=True)

                @pl.when(c == 0)
                def _():
                    pltpu.sync_copy(ones_v, shared_deg.at[idx_stage.at[0]],
                                    add=True)
                return carry2
            lax.fori_loop(0, NBA, _batch, 0)
        return carry
    lax.fori_loop(0, CPW, _chunk, 0)

    plsc.subcore_barrier()

    @pl.when(c == 0)
    def _():
        pltpu.sync_copy(shared_acc.at[pl.ds(s * NEXP, NEXP)],
                        sum_out.at[pl.ds(s * NEXP, NEXP)])
        pltpu.sync_copy(shared_deg.at[pl.ds(s * NEXP, NEXP)],
                        deg_out.at[pl.ds(s * NEXP, NEXP)])

    @pl.when(c == 1)
    def _():
        pltpu.sync_copy(shared_acc.at[pl.ds(s * NEXP, NEXP)],
                        sq_out.at[pl.ds(s * NEXP, NEXP)])


_sc_sumsq = functools.partial(
    pl.kernel,
    mesh=_MESH,
    compiler_params=_CPARAMS,
    out_type=[
        jax.ShapeDtypeStruct((NPAD, F), jnp.float32),
        jax.ShapeDtypeStruct((NPAD, F), jnp.float32),
        jax.ShapeDtypeStruct((NPAD,), jnp.float32),
    ],
    scratch_types=[
        pltpu.VMEM((CEA,), jnp.int32),
        pltpu.VMEM((CEA,), jnp.int32),
        pltpu.VMEM((G, F), jnp.float32),
        pltpu.VMEM((1, G), jnp.int32),
        pltpu.VMEM((G,), jnp.float32),
        pltpu.VMEM((NEXP,), jnp.float32),
        pltpu.VMEM_SHARED((NPAD, F), jnp.float32),
        pltpu.VMEM_SHARED((NPAD,), jnp.float32),
        pltpu.SemaphoreType.DMA,
    ],
)(_sumsq_body)


def _maxmin_body(hn_hbm, dst_hbm, src_hbm,
                 mx_out, mn_out,
                 dst_v, src_v, sel_dl, sel_src, rows_a, rows_b, mx_acc, mn_acc,
                 sem_a, sem_b):
    c = lax.axis_index("c")
    s = lax.axis_index("s")
    wid = c * 16 + s
    lo = wid * NB

    neginf = jnp.full((16,), -_F32MAX, jnp.float32)
    posinf = jnp.full((16,), _F32MAX, jnp.float32)

    def _init_row(r, carry):
        for k in range(8):
            mx_acc[r, pl.ds(k * 16, 16)] = neginf
            mn_acc[r, pl.ds(k * 16, 16)] = posinf
        return carry
    lax.fori_loop(0, NB, _init_row, 0)

    lane = lax.iota(jnp.int32, 16)

    def _chunk(ct, carry):
        ci = jnp.where(ct + wid * 2 >= NCHB, ct + wid * 2 - NCHB, ct + wid * 2)
        pltpu.sync_copy(dst_hbm.at[pl.ds(ci * CEB, CEB)], dst_v)
        pltpu.sync_copy(src_hbm.at[pl.ds(ci * CEB, CEB)], src_v)

        def _filt(g, pos):
            dv = dst_v[pl.ds(g * 16, 16)]
            sv = src_v[pl.ds(g * 16, 16)]
            dl = dv - lo
            mask = (dl >= 0) & (dl < NB)
            plsc.store_compressed(sel_dl.at[pl.ds(pos, 16)], dl, mask=mask)
            plsc.store_compressed(sel_src.at[pl.ds(pos, 16)], sv, mask=mask)
            return pos + plsc.all_reduce_population_count(mask)[0]
        cnt = lax.fori_loop(0, NGB, _filt, 0)

        nb = (cnt + G - 1) // G
        padded = nb * G
        base = (cnt // 16) * 16
        keep = (base + lane) < cnt
        zero16i = jnp.zeros((16,), jnp.int32)
        sel_src[pl.ds(base, 16)] = jnp.where(keep, sel_src[pl.ds(base, 16)],
                                             zero16i)

        def _padg(t, carry2):
            sel_src[pl.ds(base + 16 + t * 16, 16)] = zero16i
            return carry2
        lax.fori_loop(0, jnp.maximum((padded - base - 16) // 16, 0), _padg, 0)

        def _issue(b, buf, sem):
            pltpu.async_copy(hn_hbm.at[sel_src.at[pl.ds(b * G, G)]], buf, sem)

        def _wait(buf, sem):
            pltpu.make_async_copy(hn_hbm.at[sel_src.at[pl.ds(0, G)]], buf,
                                  sem).wait()

        def _edges(b, buf):
            jb = b * G
            ecnt = jnp.minimum(cnt - jb, G)

            def _edge(j, carry3):
                dl = sel_dl[pl.ds(jb + j, 16)][0]
                rows = [buf[j, pl.ds(k * 16, 16)] for k in range(8)]
                mxs = [mx_acc[dl, pl.ds(k * 16, 16)] for k in range(8)]
                mns = [mn_acc[dl, pl.ds(k * 16, 16)] for k in range(8)]
                for k in range(8):
                    mx_acc[dl, pl.ds(k * 16, 16)] = jnp.maximum(mxs[k], rows[k])
                for k in range(8):
                    mn_acc[dl, pl.ds(k * 16, 16)] = jnp.minimum(mns[k], rows[k])
                return carry3
            lax.fori_loop(0, ecnt, _edge, 0)

        @pl.when(nb > 0)
        def _():
            _issue(0, rows_a, sem_a)

        def _pair(q, carry2):
            b0 = 2 * q
            b1 = b0 + 1
            _wait(rows_a, sem_a)

            @pl.when(b1 < nb)
            def _():
                _issue(b1, rows_b, sem_b)
            _edges(b0, rows_a)

            @pl.when(b1 < nb)
            def _():
                _wait(rows_b, sem_b)

                @pl.when(b1 + 1 < nb)
                def _():
                    _issue(b1 + 1, rows_a, sem_a)
                _edges(b1, rows_b)
            return carry2
        lax.fori_loop(0, (nb + 1) // 2, _pair, 0)
        return carry
    lax.fori_loop(0, NCHB, _chunk, 0)

    pltpu.sync_copy(mx_acc, mx_out.at[pl.ds(lo, NB)])
    pltpu.sync_copy(mn_acc, mn_out.at[pl.ds(lo, NB)])


_sc_maxmin = functools.partial(
    pl.kernel,
    mesh=_MESH,
    compiler_params=_CPARAMS,
    out_type=[
        jax.ShapeDtypeStruct((NPAD, F), jnp.float32),
        jax.ShapeDtypeStruct((NPAD, F), jnp.float32),
    ],
    scratch_types=[
        pltpu.VMEM((CEB,), jnp.int32),
        pltpu.VMEM((CEB,), jnp.int32),
        pltpu.VMEM((SEL,), jnp.int32),
        pltpu.VMEM((SEL,), jnp.int32),
        pltpu.VMEM((G, F), jnp.float32),
        pltpu.VMEM((G, F), jnp.float32),
        pltpu.VMEM((NB, F), jnp.float32),
        pltpu.VMEM((NB, F), jnp.float32),
        pltpu.SemaphoreType.DMA,
        pltpu.SemaphoreType.DMA,
    ],
)(_maxmin_body)


def _mul_body(h_ref, norm_ref, o_ref):
    o_ref[...] = h_ref[...] * norm_ref[...]


def _epi_body(sum_ref, sq_ref, mx_ref, mn_ref, deg_ref, hn_ref, norm_ref,
              gamma_ref, beta_ref, o_ref):
    d = deg_ref[...]
    inv = 1.0 / d
    mean = sum_ref[...] * inv
    msq = sq_ref[...] * inv
    var = jnp.maximum(msq - mean * mean, 0.0)
    std = jnp.sqrt(var + EPS)
    a = mean + mx_ref[...] + mn_ref[...] + std
    ld = jnp.log(d + 1.0)
    scal = 1.0 + ld * (1.0 / AVG_D_LOG) + AVG_D_LOG / ld
    hcat = (hn_ref[...] + norm_ref[...] * a * scal) * (1.0 / 13.0)
    mu = jnp.mean(hcat, axis=0, keepdims=True)
    v = jnp.mean(hcat * hcat, axis=0, keepdims=True) - mu * mu
    o_ref[...] = ((hcat - mu) / jnp.sqrt(v + 1e-5)) * gamma_ref[...] + beta_ref[...]


def kernel(h, edge_index, e, norm, gamma, beta):
    src = edge_index[0]
    dst = edge_index[1]

    hn = pl.pallas_call(
        _mul_body,
        out_shape=jax.ShapeDtypeStruct((N, F), jnp.float32),
    )(h, norm)

    ssum, ssq, deg = _sc_sumsq(hn, dst, src)
    smx, smn = _sc_maxmin(hn, dst, src)

    out = pl.pallas_call(
        _epi_body,
        out_shape=jax.ShapeDtypeStruct((N, F), jnp.float32),
    )(ssum[:N], ssq[:N], smx[:N], smn[:N], deg[:N].reshape(N, 1),
      hn, norm, gamma.reshape(1, F), beta.reshape(1, F))
    return out


# speedup vs baseline: 3.8677x; 3.8677x over previous
import functools

import jax
import jax.numpy as jnp
from jax import lax
from jax.experimental import pallas as pl
from jax.experimental.pallas import tpu as pltpu
from jax.experimental.pallas import tpu_sc as plsc

N = 10000
E = 320000
F = 128
AVG_D_LOG = 3.4657359027997265
EPS = 1e-5

NPAD = 10240
NW = 32
NB = NPAD // NW
G = 64

CEA = 6400
NCHA = E // CEA
CPW = 4
NBA = CEA // G
NEXP = NPAD // 16

CEB = 16000
NCHB = E // CEB
NGB = CEB // 16
SEL = 1152

_F32MAX = 3.4028235e38

_MESH = plsc.VectorSubcoreMesh(core_axis_name="c", subcore_axis_name="s")
_CPARAMS = pltpu.CompilerParams(needs_layout_passes=False)


def _sumsq_body(hn_hbm, dst_hbm, src_hbm,
                sum_out, sq_out, deg_out,
                dst_v, src_v, rows_v, idx_stage, ones_v, zdeg_v,
                shared_acc, shared_deg, sem):
    c = lax.axis_index("c")
    s = lax.axis_index("s")

    zero16f = jnp.zeros((16,), jnp.float32)

    def _zero_rows(r, carry):
        for k in range(8):
            rows_v[r, pl.ds(k * 16, 16)] = zero16f
        return carry
    lax.fori_loop(0, G, _zero_rows, 0)
    for t in range(NEXP // G):
        pltpu.sync_copy(rows_v, shared_acc.at[pl.ds(s * NEXP + t * G, G)])
    for k in range(G // 16):
        ones_v[pl.ds(k * 16, 16)] = zero16f + 1.0
    for k in range(NEXP // 16):
        zdeg_v[pl.ds(k * 16, 16)] = zero16f
    pltpu.sync_copy(zdeg_v, shared_deg.at[pl.ds(s * NEXP, NEXP)])
    plsc.subcore_barrier()

    def _chunk(t, carry):
        ci = s + t * 16

        @pl.when(ci < NCHA)
        def _():
            pltpu.sync_copy(dst_hbm.at[pl.ds(ci * CEA, CEA)], dst_v)
            pltpu.sync_copy(src_hbm.at[pl.ds(ci * CEA, CEA)], src_v)

            def _batch(b, carry2):
                jb = b * G
                pltpu.async_copy(
                    hn_hbm.at[src_v.at[pl.ds(jb, G)]], rows_v, sem).wait()
                for k in range(G // 16):
                    idx_stage[0, pl.ds(k * 16, 16)] = dst_v[pl.ds(jb + k * 16, 16)]

                @pl.when(c == 1)
                def _():
                    def _sqr(r, carry3):
                        for k in range(8):
                            row = rows_v[r, pl.ds(k * 16, 16)]
                            rows_v[r, pl.ds(k * 16, 16)] = row * row
                        return carry3
                    lax.fori_loop(0, G, _sqr, 0)

                pltpu.sync_copy(rows_v, shared_acc.at[idx_stage.at[0]], add=True)

                @pl.when(c == 0)
                def _():
                    pltpu.sync_copy(ones_v, shared_deg.at[idx_stage.at[0]],
                                    add=True)
                return carry2
            lax.fori_loop(0, NBA, _batch, 0)
        return carry
    lax.fori_loop(0, CPW, _chunk, 0)

    plsc.subcore_barrier()

    @pl.when(c == 0)
    def _():
        pltpu.sync_copy(shared_acc.at[pl.ds(s * NEXP, NEXP)],
                        sum_out.at[pl.ds(s * NEXP, NEXP)])
        pltpu.sync_copy(shared_deg.at[pl.ds(s * NEXP, NEXP)],
                        deg_out.at[pl.ds(s * NEXP, NEXP)])

    @pl.when(c == 1)
    def _():
        pltpu.sync_copy(shared_acc.at[pl.ds(s * NEXP, NEXP)],
                        sq_out.at[pl.ds(s * NEXP, NEXP)])


_sc_sumsq = functools.partial(
    pl.kernel,
    mesh=_MESH,
    compiler_params=_CPARAMS,
    out_type=[
        jax.ShapeDtypeStruct((NPAD, F), jnp.float32),
        jax.ShapeDtypeStruct((NPAD, F), jnp.float32),
        jax.ShapeDtypeStruct((NPAD,), jnp.float32),
    ],
    scratch_types=[
        pltpu.VMEM((CEA,), jnp.int32),
        pltpu.VMEM((CEA,), jnp.int32),
        pltpu.VMEM((G, F), jnp.float32),
        pltpu.VMEM((1, G), jnp.int32),
        pltpu.VMEM((G,), jnp.float32),
        pltpu.VMEM((NEXP,), jnp.float32),
        pltpu.VMEM_SHARED((NPAD, F), jnp.float32),
        pltpu.VMEM_SHARED((NPAD,), jnp.float32),
        pltpu.SemaphoreType.DMA,
    ],
)(_sumsq_body)


def _maxmin_body(hn_hbm, dst_hbm, src_hbm,
                 mx_out, mn_out,
                 dst_v, src_v, sel_dl, sel_src, rows_a, mx_acc, mn_acc,
                 sem_a):
    c = lax.axis_index("c")
    s = lax.axis_index("s")
    wid = c * 16 + s
    lo = wid * NB

    neginf = jnp.full((16,), -_F32MAX, jnp.float32)
    posinf = jnp.full((16,), _F32MAX, jnp.float32)

    def _init_row(r, carry):
        for k in range(8):
            mx_acc[r, pl.ds(k * 16, 16)] = neginf
            mn_acc[r, pl.ds(k * 16, 16)] = posinf
        return carry
    lax.fori_loop(0, NB, _init_row, 0)

    lane = lax.iota(jnp.int32, 16)

    def _chunk(ct, carry):
        ci = lax.rem(ct + wid, NCHB)
        pltpu.sync_copy(dst_hbm.at[pl.ds(ci * CEB, CEB)], dst_v)
        pltpu.sync_copy(src_hbm.at[pl.ds(ci * CEB, CEB)], src_v)

        def _filt(g, pos):
            dv = dst_v[pl.ds(g * 16, 16)]
            sv = src_v[pl.ds(g * 16, 16)]
            dl = dv - lo
            mask = (dl >= 0) & (dl < NB)
            plsc.store_compressed(sel_dl.at[pl.ds(pos, 16)], dl, mask=mask)
            plsc.store_compressed(sel_src.at[pl.ds(pos, 16)], sv, mask=mask)
            return pos + plsc.all_reduce_population_count(mask)[0]
        cnt = lax.fori_loop(0, NGB, _filt, 0)

        nb = (cnt + G - 1) // G
        padded = nb * G
        base = (cnt // 16) * 16
        keep = (base + lane) < cnt
        zero16i = jnp.zeros((16,), jnp.int32)
        sel_src[pl.ds(base, 16)] = jnp.where(keep, sel_src[pl.ds(base, 16)],
                                             zero16i)

        def _padg(t, carry2):
            sel_src[pl.ds(base + 16 + t * 16, 16)] = zero16i
            return carry2
        lax.fori_loop(0, jnp.maximum((padded - base - 16) // 16, 0), _padg, 0)

        def _batch(b, carry2):
            jb = b * G
            pltpu.async_copy(hn_hbm.at[sel_src.at[pl.ds(jb, G)]], rows_a,
                             sem_a).wait()
            ecnt = jnp.minimum(cnt - jb, G)

            def _edge(j, carry3):
                dl = sel_dl[pl.ds(jb + j, 16)][0]
                rows = [rows_a[j, pl.ds(k * 16, 16)] for k in range(8)]
                mxs = [mx_acc[dl, pl.ds(k * 16, 16)] for k in range(8)]
                mns = [mn_acc[dl, pl.ds(k * 16, 16)] for k in range(8)]
                for k in range(8):
                    mx_acc[dl, pl.ds(k * 16, 16)] = jnp.maximum(mxs[k], rows[k])
                for k in range(8):
                    mn_acc[dl, pl.ds(k * 16, 16)] = jnp.minimum(mns[k], rows[k])
                return carry3
            lax.fori_loop(0, ecnt, _edge, 0)
            return carry2
        lax.fori_loop(0, nb, _batch, 0)
        return carry
    lax.fori_loop(0, NCHB, _chunk, 0)

    pltpu.sync_copy(mx_acc, mx_out.at[pl.ds(lo, NB)])
    pltpu.sync_copy(mn_acc, mn_out.at[pl.ds(lo, NB)])


_sc_maxmin = functools.partial(
    pl.kernel,
    mesh=_MESH,
    compiler_params=_CPARAMS,
    out_type=[
        jax.ShapeDtypeStruct((NPAD, F), jnp.float32),
        jax.ShapeDtypeStruct((NPAD, F), jnp.float32),
    ],
    scratch_types=[
        pltpu.VMEM((CEB,), jnp.int32),
        pltpu.VMEM((CEB,), jnp.int32),
        pltpu.VMEM((SEL,), jnp.int32),
        pltpu.VMEM((SEL,), jnp.int32),
        pltpu.VMEM((G, F), jnp.float32),
        pltpu.VMEM((NB, F), jnp.float32),
        pltpu.VMEM((NB, F), jnp.float32),
        pltpu.SemaphoreType.DMA,
    ],
)(_maxmin_body)


def _mul_body(h_ref, norm_ref, o_ref):
    o_ref[...] = h_ref[...] * norm_ref[...]


def _epi_body(sum_ref, sq_ref, mx_ref, mn_ref, deg_ref, hn_ref, norm_ref,
              gamma_ref, beta_ref, o_ref):
    d = deg_ref[...]
    inv = 1.0 / d
    mean = sum_ref[...] * inv
    msq = sq_ref[...] * inv
    var = jnp.maximum(msq - mean * mean, 0.0)
    std = jnp.sqrt(var + EPS)
    a = mean + mx_ref[...] + mn_ref[...] + std
    ld = jnp.log(d + 1.0)
    scal = 1.0 + ld * (1.0 / AVG_D_LOG) + AVG_D_LOG / ld
    hcat = (hn_ref[...] + norm_ref[...] * a * scal) * (1.0 / 13.0)
    mu = jnp.mean(hcat, axis=0, keepdims=True)
    v = jnp.mean(hcat * hcat, axis=0, keepdims=True) - mu * mu
    o_ref[...] = ((hcat - mu) / jnp.sqrt(v + 1e-5)) * gamma_ref[...] + beta_ref[...]


def kernel(h, edge_index, e, norm, gamma, beta):
    src = edge_index[0]
    dst = edge_index[1]

    hn = pl.pallas_call(
        _mul_body,
        out_shape=jax.ShapeDtypeStruct((N, F), jnp.float32),
    )(h, norm)

    ssum, ssq, deg = _sc_sumsq(hn, dst, src)
    smx, smn = _sc_maxmin(hn, dst, src)

    out = pl.pallas_call(
        _epi_body,
        out_shape=jax.ShapeDtypeStruct((N, F), jnp.float32),
    )(ssum[:N], ssq[:N], smx[:N], smn[:N], deg[:N].reshape(N, 1),
      hn, norm, gamma.reshape(1, F), beta.reshape(1, F))
    return out
